# baseline (speedup 1.0000x reference)
import jax
import jax.numpy as jnp
from jax import lax
from jax.experimental import pallas as pl
from jax.experimental.pallas import tpu as pltpu
from jax.experimental.pallas import tpu_sc as plsc

N = 10000
DF = 128
E_RAW = 320000
E_TOT = E_RAW + N
E_PAD = 331776
NW = 32
N_ACC = 10016
RPT = N_ACC // 16

_HIGH = lax.Precision.HIGHEST


def _mesh():
    return plsc.VectorSubcoreMesh(core_axis_name="c", subcore_axis_name="s")


_SC_PARAMS = pltpu.CompilerParams(use_tc_tiling_on_sc=False,
                                  needs_layout_passes=False)


def _vperm(v, idx):
    return lax.gather(
        v, idx.reshape(16, 1),
        lax.GatherDimensionNumbers(offset_dims=(), collapsed_slice_dims=(0,),
                                   start_index_map=(0,)),
        (1,), mode=lax.GatherScatterMode.PROMISE_IN_BOUNDS)


def _sc_layer0(q, k, v, row, row_s, col, z8, z64):
    ca = 144
    per_w = E_PAD // NW
    n_ch = per_w // ca

    i32 = jnp.int32
    f32 = jnp.float32
    scr = ([pltpu.VMEM((ca,), i32)] * 6 +
           [pltpu.VMEM((ca, 64), f32)] * 7 +
           [pltpu.VMEM((ca, 8), f32)] +
           [pltpu.VMEM_SHARED((N_ACC, 8), f32),
            pltpu.VMEM_SHARED((N_ACC, 64), f32)] +
           [pltpu.SemaphoreType.DMA] * 6)

    @pl.kernel(out_type=(jax.ShapeDtypeStruct((2, N_ACC, 64), f32),
                         jax.ShapeDtypeStruct((2, N_ACC, 8), f32)),
               mesh=_mesh(), compiler_params=_SC_PARAMS, scratch_types=scr)
    def l0_kernel(q_hbm, k_hbm, v_hbm, row_hbm, rs_hbm, col_hbm,
                  z8_hbm, z64_hbm, out_hbm, dh_hbm,
                  ir0, ir1, is0, is1, ic0, ic1,
                  qe0, qe1, ke0, ke1, ve0, ve1, ct_v, pv_v,
                  den, acc, sq0, sq1, sk0, sk1, sv0, sv1):
        c = lax.axis_index("c")
        s = lax.axis_index("s")
        base_w = (s * 2 + c) * per_w
        IR = (ir0, ir1)
        IS = (is0, is1)
        IC = (ic0, ic1)
        QE = (qe0, qe1)
        KE = (ke0, ke1)
        VE = (ve0, ve1)
        SQ = (sq0, sq1)
        SK = (sk0, sk1)
        SV = (sv0, sv1)
        io16 = lax.iota(i32, 16)
        p8x = io16 ^ 8
        h8 = io16 & 7
        lo8 = io16 < 8
        hi1 = jnp.where(io16 >= 8, 1, 0).astype(i32)

        rows = pl.ds(s * RPT, RPT)
        pltpu.sync_copy(z8_hbm, den.at[rows])
        pltpu.sync_copy(z64_hbm, acc.at[rows])
        plsc.subcore_barrier()

        def issue(b, ch):
            base = base_w + ch * ca
            pltpu.sync_copy(row_hbm.at[pl.ds(base, ca)], IR[b])
            pltpu.sync_copy(rs_hbm.at[pl.ds(base, ca)], IS[b])
            pltpu.sync_copy(col_hbm.at[pl.ds(base, ca)], IC[b])
            pltpu.async_copy(q_hbm.at[IR[b]], QE[b], SQ[b])
            pltpu.async_copy(k_hbm.at[IC[b]], KE[b], SK[b])
            pltpu.async_copy(v_hbm.at[IC[b]], VE[b], SV[b])

        for b in range(2):
            issue(b, b)

        @pl.loop(0, n_ch // 2)
        def _(gg):
            for b in range(2):
                ch = gg * 2 + b
                pltpu.make_async_copy(q_hbm.at[IR[b]], QE[b], SQ[b]).wait()
                pltpu.make_async_copy(k_hbm.at[IC[b]], KE[b], SK[b]).wait()
                pltpu.make_async_copy(v_hbm.at[IC[b]], VE[b], SV[b]).wait()

                qe_v, ke_v, ve_v = QE[b], KE[b], VE[b]

                @plsc.parallel_loop(0, ca, unroll=8)
                def _(e):
                    acc_r = None
                    for j in range(4):
                        qv = qe_v[e, pl.ds(j * 16, 16)]
                        kv = ke_v[e, pl.ds(j * 16, 16)]
                        pr = qv * kv
                        acc_r = pr if acc_r is None else acc_r + pr
                    att = acc_r + _vperm(acc_r, p8x)
                    t = jnp.exp(att)
                    plsc.store_scatter(pv_v, [jnp.full((16,), e, i32), h8],
                                       t, mask=lo8)
                    for j in range(4):
                        tj = _vperm(t, jnp.full((16,), 2 * j, i32) + hi1)
                        sl = pl.ds(j * 16, 16)
                        ct_v[e, sl] = ve_v[e, sl] * tj

                pltpu.sync_copy(pv_v, den.at[IS[b]], add=True)
                pltpu.sync_copy(ct_v, acc.at[IS[b]], add=True)

                @pl.when(ch + 2 < n_ch)
                def _():
                    issue(b, ch + 2)

        plsc.subcore_barrier()
        pltpu.sync_copy(den.at[rows], dh_hbm.at[c, rows])
        pltpu.sync_copy(acc.at[rows], out_hbm.at[c, rows])

    return l0_kernel(q, k, v, row, row_s, col, z8, z64)


def _sc_layer1(q1t, k1t, v1t, row, row_s, col, z8):
    ca = 1296
    per_w = E_PAD // NW
    n_ch = per_w // ca

    i32 = jnp.int32
    f32 = jnp.float32
    scr = ([pltpu.VMEM((ca,), i32)] * 6 +
           [pltpu.VMEM((ca, 8), f32)] * 8 +
           [pltpu.VMEM_SHARED((N_ACC, 8), f32),
            pltpu.VMEM_SHARED((N_ACC, 8), f32)] +
           [pltpu.SemaphoreType.DMA] * 6)

    @pl.kernel(out_type=(jax.ShapeDtypeStruct((2, N_ACC, 8), f32),
                         jax.ShapeDtypeStruct((2, N_ACC, 8), f32)),
               mesh=_mesh(), compiler_params=_SC_PARAMS, scratch_types=scr)
    def l1_kernel(q_hbm, k_hbm, v_hbm, row_hbm, rs_hbm, col_hbm,
                  z8_hbm, out_hbm, dh_hbm,
                  ir0, ir1, is0, is1, ic0, ic1,
                  qe0, qe1, ke0, ke1, ve0, ve1, ct_v, pv_v,
                  den, acc, sq0, sq1, sk0, sk1, sv0, sv1):
        c = lax.axis_index("c")
        s = lax.axis_index("s")
        base_w = (s * 2 + c) * per_w
        IR = (ir0, ir1)
        IS = (is0, is1)
        IC = (ic0, ic1)
        QE = (qe0, qe1)
        KE = (ke0, ke1)
        VE = (ve0, ve1)
        SQ = (sq0, sq1)
        SK = (sk0, sk1)
        SV = (sv0, sv1)
        io16 = lax.iota(i32, 16)
        h8 = io16 & 7
        hi1 = jnp.where(io16 >= 8, 1, 0).astype(i32)

        rows = pl.ds(s * RPT, RPT)
        pltpu.sync_copy(z8_hbm, den.at[rows])
        pltpu.sync_copy(z8_hbm, acc.at[rows])
        plsc.subcore_barrier()

        def issue(b, ch):
            base = base_w + ch * ca
            pltpu.sync_copy(row_hbm.at[pl.ds(base, ca)], IR[b])
            pltpu.sync_copy(rs_hbm.at[pl.ds(base, ca)], IS[b])
            pltpu.sync_copy(col_hbm.at[pl.ds(base, ca)], IC[b])
            pltpu.async_copy(q_hbm.at[IR[b]], QE[b], SQ[b])
            pltpu.async_copy(k_hbm.at[IC[b]], KE[b], SK[b])
            pltpu.async_copy(v_hbm.at[IC[b]], VE[b], SV[b])

        for b in range(2):
            issue(b, b)

        @pl.loop(0, n_ch // 2)
        def _(gg):
            for b in range(2):
                ch = gg * 2 + b
                pltpu.make_async_copy(q_hbm.at[IR[b]], QE[b], SQ[b]).wait()
                pltpu.make_async_copy(k_hbm.at[IC[b]], KE[b], SK[b]).wait()
                pltpu.make_async_copy(v_hbm.at[IC[b]], VE[b], SV[b]).wait()

                qe_v, ke_v, ve_v = QE[b], KE[b], VE[b]

                @plsc.parallel_loop(0, ca // 2, unroll=8)
                def _(ee):
                    e2 = jnp.full((16,), 2 * ee, i32) + hi1
                    qv = plsc.load_gather(qe_v, [e2, h8])
                    kv = plsc.load_gather(ke_v, [e2, h8])
                    vv = plsc.load_gather(ve_v, [e2, h8])
                    t = jnp.exp(qv * kv)
                    plsc.store_scatter(pv_v, [e2, h8], t)
                    plsc.store_scatter(ct_v, [e2, h8], t * vv)

                pltpu.sync_copy(pv_v, den.at[IS[b]], add=True)
                pltpu.sync_copy(ct_v, acc.at[IS[b]], add=True)

                @pl.when(ch + 2 < n_ch)
                def _():
                    issue(b, ch + 2)

        plsc.subcore_barrier()
        pltpu.sync_copy(den.at[rows], dh_hbm.at[c, rows])
        pltpu.sync_copy(acc.at[rows], out_hbm.at[c, rows])

    return l1_kernel(q1t, k1t, v1t, row, row_s, col, z8)


def _tc_qkv(x, Wq, bq, Wk, bk, Wv):
    blk = 2000

    def body(x_ref, wq_ref, bq_ref, wk_ref, bk_ref, wv_ref, q_ref, k_ref, v_ref):
        xb = x_ref[...]
        q_ref[...] = jax.nn.relu(
            jnp.dot(xb, wq_ref[...], preferred_element_type=jnp.float32,
                    precision=_HIGH) + bq_ref[...])
        k_ref[...] = jax.nn.relu(
            jnp.dot(xb, wk_ref[...], preferred_element_type=jnp.float32,
                    precision=_HIGH) + bk_ref[...])
        v_ref[...] = jnp.dot(xb, wv_ref[...], preferred_element_type=jnp.float32,
                             precision=_HIGH)

    full = lambda i: (0, 0)
    o64 = jax.ShapeDtypeStruct((N, 64), jnp.float32)
    return pl.pallas_call(
        body,
        grid=(N // blk,),
        in_specs=[
            pl.BlockSpec((blk, DF), lambda i: (i, 0)),
            pl.BlockSpec((DF, 64), full),
            pl.BlockSpec((1, 64), full),
            pl.BlockSpec((DF, 64), full),
            pl.BlockSpec((1, 64), full),
            pl.BlockSpec((DF, 64), full),
        ],
        out_specs=[pl.BlockSpec((blk, 64), lambda i: (i, 0))] * 3,
        out_shape=[o64, o64, o64],
    )(x, Wq, bq.reshape(1, 64), Wk, bk.reshape(1, 64), Wv)


def _tc_layer1_tables(oa, ob, dparts, b0, Wq1b, bq1b, Wk1b, bk1b, W1p):
    blk = 2000

    def body(a_ref, b_ref, dp_ref, b0_ref, wq_ref, bq_ref, wk_ref, bk_ref,
             wv_ref, q_ref, k_ref, v_ref):
        d = dp_ref[0] + dp_ref[1] + 1e-16
        d64 = jnp.broadcast_to(d[:, :, None], (blk, 8, 8)).reshape(blk, 64)
        h = jax.nn.relu((a_ref[...] + b_ref[...]) / d64 + b0_ref[...])
        q_ref[...] = jax.nn.relu(
            jnp.dot(h, wq_ref[...], preferred_element_type=jnp.float32,
                    precision=_HIGH) + bq_ref[...])
        k_ref[...] = jax.nn.relu(
            jnp.dot(h, wk_ref[...], preferred_element_type=jnp.float32,
                    precision=_HIGH) + bk_ref[...])
        v_ref[...] = jnp.dot(h, wv_ref[...], preferred_element_type=jnp.float32,
                             precision=_HIGH)

    full = lambda i: (0, 0)
    o8 = jax.ShapeDtypeStruct((N, 8), jnp.float32)
    return pl.pallas_call(
        body,
        grid=(N // blk,),
        in_specs=[
            pl.BlockSpec((blk, 64), lambda i: (i, 0)),
            pl.BlockSpec((blk, 64), lambda i: (i, 0)),
            pl.BlockSpec((2, blk, 8), lambda i: (0, i, 0)),
            pl.BlockSpec((1, 64), full),
            pl.BlockSpec((64, 8), full),
            pl.BlockSpec((1, 8), full),
            pl.BlockSpec((64, 8), full),
            pl.BlockSpec((1, 8), full),
            pl.BlockSpec((64, 8), full),
        ],
        out_specs=[pl.BlockSpec((blk, 8), lambda i: (i, 0))] * 3,
        out_shape=[o8, o8, o8],
    )(oa, ob, dparts, b0.reshape(1, 64), Wq1b, bq1b, Wk1b, bk1b, W1p)


def _tc_final(o1a, o1b, dparts, b1p):
    blk = 2000

    def body(a_ref, b_ref, dp_ref, bias_ref, o_ref):
        d = dp_ref[0] + dp_ref[1] + 1e-16
        o_ref[...] = (a_ref[...] + b_ref[...]) / d + bias_ref[...]

    return pl.pallas_call(
        body,
        grid=(N // blk,),
        in_specs=[
            pl.BlockSpec((blk, 8), lambda i: (i, 0)),
            pl.BlockSpec((blk, 8), lambda i: (i, 0)),
            pl.BlockSpec((2, blk, 8), lambda i: (0, i, 0)),
            pl.BlockSpec((1, 8), lambda i: (0, 0)),
        ],
        out_specs=pl.BlockSpec((blk, 8), lambda i: (i, 0)),
        out_shape=jax.ShapeDtypeStruct((N, 8), jnp.float32),
    )(o1a, o1b, dparts, b1p)


def kernel(x, edge_index, Wq0, bq0, Wk0, bk0, W0, b0, Wq1, bq1, Wk1, bk1, W1, b1):
    loops = jnp.arange(N, dtype=jnp.int32)
    pad = jnp.zeros((E_PAD - E_TOT,), jnp.int32)
    row = jnp.concatenate([edge_index[0].astype(jnp.int32), loops, pad])
    col = jnp.concatenate([edge_index[1].astype(jnp.int32), loops, pad])
    row_s = jnp.concatenate([edge_index[0].astype(jnp.int32), loops,
                             jnp.full((E_PAD - E_TOT,), N, jnp.int32)])

    z8 = jnp.zeros((RPT, 8), jnp.float32)
    z64 = jnp.zeros((RPT, 64), jnp.float32)

    perm = (jnp.arange(64) % 8) * 8 + jnp.arange(64) // 8
    Q, K, V = _tc_qkv(x, Wq0[:, perm], bq0[perm], Wk0[:, perm], bk0[perm], W0)
    opart, dpart = _sc_layer0(Q, K, V, row, row_s, col, z8, z64)

    Wq1b = jnp.broadcast_to(Wq1, (64, 8))
    bq1b = jnp.broadcast_to(bq1, (1, 8))
    Wk1b = jnp.broadcast_to(Wk1, (64, 8))
    bk1b = jnp.broadcast_to(bk1, (1, 8))
    W1p = jnp.pad(W1, ((0, 0), (0, 1)))
    q1t, k1t, v1t = _tc_layer1_tables(opart[0], opart[1], dpart, b0,
                                      Wq1b, bq1b, Wk1b, bk1b, W1p)
    o1part, d1part = _sc_layer1(q1t, k1t, v1t, row, row_s, col, z8)
    out8 = _tc_final(o1part[0], o1part[1], d1part,
                     jnp.pad(b1, (0, 1)).reshape(1, 8))
    return out8[:, :7]
